# baseline (speedup 1.0000x reference)
import jax
import jax.numpy as jnp
from jax import lax
from jax.experimental import pallas as pl
from jax.experimental.pallas import tpu as pltpu
from jax.experimental.pallas import tpu_sc as plsc

D_MODEL = 768
D_SAE = 12288
BATCH = 128
AUX_COEF = 0.03125
BLK = 1024
NBLK = D_SAE // BLK
MININT = -2147483648
NC, NS = 2, 16
NWORK = NC * NS
ROWS_PER_TILE = BATCH // NWORK
NVREG = D_SAE // 16
def _ukeys(v):
    b = lax.bitcast_convert_type(v, jnp.int32)
    m = b >> 31
    return lax.bitcast_convert_type(b ^ (m | jnp.int32(-2147483648)), jnp.uint32)


def _encode_body(x_ref, w_ref, b_ref, pre_ref):
    pre_ref[...] = (
        jnp.dot(x_ref[...], w_ref[...], preferred_element_type=jnp.float32)
        + b_ref[...]
    )


def _encode(x, W_enc, b_enc2d):
    return pl.pallas_call(
        _encode_body,
        grid=(NBLK,),
        in_specs=[
            pl.BlockSpec((BATCH, D_MODEL), lambda i: (0, 0)),
            pl.BlockSpec((D_MODEL, BLK), lambda i: (0, i)),
            pl.BlockSpec((1, BLK), lambda i: (0, i)),
        ],
        out_specs=pl.BlockSpec((BATCH, BLK), lambda i: (0, i)),
        out_shape=jax.ShapeDtypeStruct((BATCH, D_SAE), jnp.float32),
    )(x, W_enc, b_enc2d)


def _sc_refine(cand_ref, n, bkt, rank):
    prefix = lax.shift_left(bkt, 23)
    nv = n // 16 + 1

    def bitloop(i, T):
        cand_t = T | lax.shift_left(jnp.int32(1), 22 - i)

        def cnt_body(j, acc):
            u = cand_ref[pl.ds(j * 16, 16)]
            return acc + jnp.where(u >= cand_t, 1, 0).astype(jnp.int32)

        acc = lax.fori_loop(0, nv, cnt_body, jnp.zeros((16,), jnp.int32))
        return jnp.where(jnp.sum(acc) >= rank, cand_t, T)

    return lax.fori_loop(0, 23, bitloop, prefix)


def _sc_body(pre_hbm, out_hbm, rows_v, hist_v, suf_v, candA_v, candB_v,
             outbuf_v, sem):
    wid = lax.axis_index("s") * NC + lax.axis_index("c")
    base = wid * ROWS_PER_TILE
    pltpu.sync_copy(pre_hbm.at[pl.ds(base, ROWS_PER_TILE)], rows_v)

    lane = lax.iota(jnp.int32, 16)
    lane512 = lane * 512
    zeros16 = jnp.zeros((16,), jnp.int32)
    ones16 = jnp.ones((16,), jnp.int32)
    minint_v = jnp.full((16,), MININT, jnp.int32)

    def zh(i, _):
        hist_v[pl.ds(i * 16, 16)] = zeros16
        return 0

    lax.fori_loop(0, 512, zh, 0)
    suf_v[pl.ds(512, 16)] = zeros16
    suf_v[pl.ds(528, 16)] = zeros16

    outvec = zeros16
    for r in range(ROWS_PER_TILE):
        def hbody(i, _):
            v = rows_v[r, pl.ds(i * 16, 16)]
            b = lax.bitcast_convert_type(v, jnp.int32)
            u = b ^ ((b >> 31) | jnp.int32(MININT))
            bucket = lax.shift_right_logical(u, 23)
            plsc.addupdate_scatter(hist_v, [lane512 + bucket], ones16)
            return 0

        lax.fori_loop(0, NVREG, hbody, 0)

        def mbody(i, carry):
            c = 31 - i
            acc = zeros16
            for l in range(16):
                off = l * 512 + c * 16
                acc = acc + hist_v[pl.ds(off, 16)]
                hist_v[pl.ds(off, 16)] = zeros16
            s = lax.rev(jnp.cumsum(lax.rev(acc, (0,)), axis=0), (0,)) + carry
            suf_v[pl.ds(c * 16, 16)] = s
            return jnp.max(s)

        lax.fori_loop(0, 32, mbody, jnp.int32(0))

        def fbody(i, carry):
            a32, a64 = carry
            chunk = suf_v[pl.ds(i * 16, 16)]
            bidx = lane + i * 16
            a32 = jnp.maximum(a32, jnp.where(chunk >= 32, bidx, -1))
            a64 = jnp.maximum(a64, jnp.where(chunk >= 64, bidx, -1))
            return a32, a64

        neg1 = jnp.full((16,), -1, jnp.int32)
        a32, a64 = lax.fori_loop(0, 32, fbody, (neg1, neg1))
        b32 = jnp.max(a32)
        b64 = jnp.max(a64)
        c32_above = jnp.max(plsc.load_gather(suf_v, [zeros16 + (b32 + 1)]))
        c64_above = jnp.max(plsc.load_gather(suf_v, [zeros16 + (b64 + 1)]))

        def cbody(i, carry):
            oA, oB = carry
            v = rows_v[r, pl.ds(i * 16, 16)]
            b = lax.bitcast_convert_type(v, jnp.int32)
            u = b ^ ((b >> 31) | jnp.int32(MININT))
            bucket = lax.shift_right_logical(u, 23)
            mA = bucket == b32
            mB = bucket == b64
            plsc.store_compressed(candA_v.at[pl.ds(oA, 16)], u, mask=mA)
            plsc.store_compressed(candB_v.at[pl.ds(oB, 16)], u, mask=mB)
            oA = oA + jnp.max(plsc.all_reduce_population_count(mA))
            oB = oB + jnp.max(plsc.all_reduce_population_count(mB))
            return oA, oB

        nA, nB = lax.fori_loop(0, NVREG, cbody, (jnp.int32(0), jnp.int32(0)))
        candA_v[pl.ds(nA, 16)] = minint_v
        candB_v[pl.ds(nB, 16)] = minint_v

        T32 = _sc_refine(candA_v, nA, b32, 32 - c32_above)
        T64 = _sc_refine(candB_v, nB, b64, 64 - c64_above)
        outvec = jnp.where(lane == r, T32, outvec)
        outvec = jnp.where(lane == (4 + r), T64, outvec)

    outbuf_v[...] = outvec
    pltpu.sync_copy(outbuf_v, out_hbm.at[wid])


def _thresholds(pre):
    mesh = plsc.VectorSubcoreMesh(core_axis_name="c", subcore_axis_name="s",
                                  num_cores=NC, num_subcores=NS)
    out = pl.kernel(
        _sc_body,
        out_type=jax.ShapeDtypeStruct((NWORK, 16), jnp.int32),
        mesh=mesh,
        scratch_types=[
            pltpu.VMEM((ROWS_PER_TILE, D_SAE), jnp.float32),
            pltpu.VMEM((16 * 512,), jnp.int32),
            pltpu.VMEM((544,), jnp.int32),
            pltpu.VMEM((D_SAE + 32,), jnp.int32),
            pltpu.VMEM((D_SAE + 32,), jnp.int32),
            pltpu.VMEM((16,), jnp.int32),
            pltpu.SemaphoreType.DMA,
        ],
        compiler_params=pltpu.CompilerParams(needs_layout_passes=False),
    )(pre)
    t32 = out[:, 0:4].reshape(BATCH, 1)
    t64 = out[:, 4:8].reshape(BATCH, 1)
    return (lax.bitcast_convert_type(t32, jnp.uint32),
            lax.bitcast_convert_type(t64, jnp.uint32))


def _decode_body(pre_ref, t32_ref, t64_ref, wd_ref, x_ref, bdec_ref,
                 lat_ref, xhat_ref, loss_ref, aux_ref, acc1, acc2):
    i = pl.program_id(0)

    @pl.when(i == 0)
    def _():
        acc1[...] = jnp.zeros_like(acc1)
        acc2[...] = jnp.zeros_like(acc2)

    pre = pre_ref[...]
    u = _ukeys(pre)
    relu = jnp.maximum(pre, 0.0)
    m1 = u >= t32_ref[...]
    m2 = (u >= t64_ref[...]) & jnp.logical_not(m1)
    lat = jnp.where(m1, relu, 0.0)
    lat_ref[...] = lat

    wd = wd_ref[...]
    norm2 = jnp.sum(wd * wd, axis=1, keepdims=True)
    inv = 1.0 / jnp.maximum(jnp.sqrt(norm2), 1e-12)
    wdn = wd * inv
    acc1[...] += jnp.dot(lat, wdn, preferred_element_type=jnp.float32)
    aux = jnp.where(m2, relu, 0.0)
    acc2[...] += jnp.dot(aux, wdn, preferred_element_type=jnp.float32)

    @pl.when(i == NBLK - 1)
    def _():
        xh = acc1[...] + bdec_ref[...]
        xhat_ref[...] = xh
        d = xh - x_ref[...]
        loss_ref[0, 0] = jnp.mean(d * d)
        a = acc2[...] + d
        aux_ref[0, 0] = AUX_COEF * jnp.mean(a * a)


def _decode(pre, t32, t64, W_dec, x, bdec2d):
    return pl.pallas_call(
        _decode_body,
        grid=(NBLK,),
        in_specs=[
            pl.BlockSpec((BATCH, BLK), lambda i: (0, i)),
            pl.BlockSpec((BATCH, 1), lambda i: (0, 0)),
            pl.BlockSpec((BATCH, 1), lambda i: (0, 0)),
            pl.BlockSpec((BLK, D_MODEL), lambda i: (i, 0)),
            pl.BlockSpec((BATCH, D_MODEL), lambda i: (0, 0)),
            pl.BlockSpec((1, D_MODEL), lambda i: (0, 0)),
        ],
        out_specs=(
            pl.BlockSpec((BATCH, BLK), lambda i: (0, i)),
            pl.BlockSpec((BATCH, D_MODEL), lambda i: (0, 0)),
            pl.BlockSpec(memory_space=pltpu.SMEM, block_shape=(1, 1),
                         index_map=lambda i: (0, 0)),
            pl.BlockSpec(memory_space=pltpu.SMEM, block_shape=(1, 1),
                         index_map=lambda i: (0, 0)),
        ),
        out_shape=(
            jax.ShapeDtypeStruct((BATCH, D_SAE), jnp.float32),
            jax.ShapeDtypeStruct((BATCH, D_MODEL), jnp.float32),
            jax.ShapeDtypeStruct((1, 1), jnp.float32),
            jax.ShapeDtypeStruct((1, 1), jnp.float32),
        ),
        scratch_shapes=[
            pltpu.VMEM((BATCH, D_MODEL), jnp.float32),
            pltpu.VMEM((BATCH, D_MODEL), jnp.float32),
        ],
    )(pre, t32, t64, W_dec, x, bdec2d)


def kernel(x, W_enc, b_enc, W_dec, b_dec):
    pre = _encode(x, W_enc, b_enc.reshape(1, D_SAE))
    t32, t64 = _thresholds(pre)
    latents, x_hat, loss, aux_loss = _decode(
        pre, t32, t64, W_dec, x, b_dec.reshape(1, D_MODEL))
    return x_hat, latents, loss[0, 0], aux_loss[0, 0]


# speedup vs baseline: 1.1776x; 1.1776x over previous
import jax
import jax.numpy as jnp
from jax import lax
from jax.experimental import pallas as pl
from jax.experimental.pallas import tpu as pltpu
from jax.experimental.pallas import tpu_sc as plsc

D_MODEL = 768
D_SAE = 12288
BATCH = 128
AUX_COEF = 0.03125
BLK = 1024
NBLK = D_SAE // BLK
MININT = -2147483648
NC, NS = 2, 16
NWORK = NC * NS
ROWS_PER_TILE = BATCH // NWORK
NVREG = D_SAE // 16
def _skeys(v):
    b = lax.bitcast_convert_type(v, jnp.int32)
    return b ^ ((b >> 31) & jnp.int32(0x7FFFFFFF))


def _encode_body(x_ref, w_ref, b_ref, pre_ref):
    pre_ref[...] = (
        jnp.dot(x_ref[...], w_ref[...], preferred_element_type=jnp.float32)
        + b_ref[...]
    )


def _encode(x, W_enc, b_enc2d):
    return pl.pallas_call(
        _encode_body,
        grid=(NBLK,),
        in_specs=[
            pl.BlockSpec((BATCH, D_MODEL), lambda i: (0, 0)),
            pl.BlockSpec((D_MODEL, BLK), lambda i: (0, i)),
            pl.BlockSpec((1, BLK), lambda i: (0, i)),
        ],
        out_specs=pl.BlockSpec((BATCH, BLK), lambda i: (0, i)),
        out_shape=jax.ShapeDtypeStruct((BATCH, D_SAE), jnp.float32),
    )(x, W_enc, b_enc2d)


def _sc_refine(cand_ref, nv, bkt, rank):
    prefix = lax.shift_left(bkt, 23) ^ jnp.int32(MININT)

    def bitloop(i, T):
        cand_t = T | lax.shift_left(jnp.int32(1), 22 - i)

        def cnt_body(j, acc):
            for k in range(4):
                u = cand_ref[pl.ds(j * 64 + k * 16, 16)]
                acc = acc + jnp.where(u >= cand_t, 1, 0).astype(jnp.int32)
            return acc

        acc = lax.fori_loop(0, nv, cnt_body, jnp.zeros((16,), jnp.int32))
        return jnp.where(jnp.sum(acc) >= rank, cand_t, T)

    return lax.fori_loop(0, 23, bitloop, prefix)


def _sc_body(pre_hbm, out_hbm, rows_v, hist_v, suf_v, cand_v, outbuf_v, sem):
    wid = lax.axis_index("s") * NC + lax.axis_index("c")
    base = wid * ROWS_PER_TILE
    cp = pltpu.async_copy(pre_hbm.at[pl.ds(base, 1)],
                          rows_v.at[pl.ds(0, 1)], sem)

    lane = lax.iota(jnp.int32, 16)
    zeros16 = jnp.zeros((16,), jnp.int32)
    ones16 = jnp.ones((16,), jnp.int32)
    minint = jnp.int32(MININT)
    minint_v = jnp.full((16,), MININT, jnp.int32)

    def zh(i, _):
        hist_v[pl.ds(i * 16, 16)] = zeros16
        return 0

    lax.fori_loop(0, 32, zh, 0)
    suf_v[pl.ds(512, 16)] = zeros16
    suf_v[pl.ds(528, 16)] = zeros16

    outvec = zeros16
    for r in range(ROWS_PER_TILE):
        cp.wait()
        if r + 1 < ROWS_PER_TILE:
            cp = pltpu.async_copy(pre_hbm.at[pl.ds(base + r + 1, 1)],
                                  rows_v.at[pl.ds(r + 1, 1)], sem)

        def hbody(i, _):
            for k in range(8):
                v = rows_v[r, pl.ds(i * 128 + k * 16, 16)]
                sk = _skeys(v)
                bucket = lax.shift_right_logical(sk ^ minint, 23)
                plsc.addupdate_scatter(hist_v, [bucket], ones16)
            return 0

        lax.fori_loop(0, NVREG // 8, hbody, 0)

        def sbody(i, carry):
            c = 31 - i
            chunk = hist_v[pl.ds(c * 16, 16)]
            hist_v[pl.ds(c * 16, 16)] = zeros16
            s = lax.rev(jnp.cumsum(lax.rev(chunk, (0,)), axis=0), (0,)) + carry
            suf_v[pl.ds(c * 16, 16)] = s
            return jnp.max(s)

        lax.fori_loop(0, 32, sbody, jnp.int32(0))

        def fbody(i, carry):
            a32, a64 = carry
            chunk = suf_v[pl.ds(i * 16, 16)]
            bidx = lane + i * 16
            a32 = jnp.maximum(a32, jnp.where(chunk >= 32, bidx, -1))
            a64 = jnp.maximum(a64, jnp.where(chunk >= 64, bidx, -1))
            return a32, a64

        neg1 = jnp.full((16,), -1, jnp.int32)
        a32, a64 = lax.fori_loop(0, 32, fbody, (neg1, neg1))
        b32 = jnp.max(a32)
        b64 = jnp.max(a64)
        c32_above = jnp.max(plsc.load_gather(suf_v, [zeros16 + (b32 + 1)]))
        c64_above = jnp.max(plsc.load_gather(suf_v, [zeros16 + (b64 + 1)]))
        n32b = jnp.max(plsc.load_gather(suf_v, [zeros16 + b32])) - c32_above

        def cbody(i, off):
            for k in range(4):
                v = rows_v[r, pl.ds(i * 64 + k * 16, 16)]
                sk = _skeys(v)
                bucket = lax.shift_right_logical(sk ^ minint, 23)
                mAB = (bucket == b32) | (bucket == b64)
                pc = plsc.all_reduce_population_count(mAB)
                csum = plsc.cumsum(mAB.astype(jnp.int32))
                plsc.store_scatter(cand_v, [off + csum - 1], sk, mask=mAB)
                off = off + pc
            return off

        off = lax.fori_loop(0, NVREG // 4, cbody, zeros16)
        nAB = jnp.max(off)
        for t in range(4):
            plsc.store_scatter(cand_v, [nAB + lane + t * 16], minint_v)
        nv = nAB // 64 + 1

        rank32 = 32 - c32_above
        rank64 = (64 - c64_above) + jnp.where(b64 == b32, 0, n32b)
        T32 = _sc_refine(cand_v, nv, b32, rank32)
        T64 = _sc_refine(cand_v, nv, b64, rank64)
        outvec = jnp.where(lane == r, T32, outvec)
        outvec = jnp.where(lane == (4 + r), T64, outvec)

    outbuf_v[...] = outvec
    pltpu.sync_copy(outbuf_v, out_hbm.at[wid])


def _thresholds(pre):
    mesh = plsc.VectorSubcoreMesh(core_axis_name="c", subcore_axis_name="s",
                                  num_cores=NC, num_subcores=NS)
    out = pl.kernel(
        _sc_body,
        out_type=jax.ShapeDtypeStruct((NWORK, 16), jnp.int32),
        mesh=mesh,
        scratch_types=[
            pltpu.VMEM((ROWS_PER_TILE, D_SAE), jnp.float32),
            pltpu.VMEM((512,), jnp.int32),
            pltpu.VMEM((544,), jnp.int32),
            pltpu.VMEM((D_SAE + 64,), jnp.int32),
            pltpu.VMEM((16,), jnp.int32),
            pltpu.SemaphoreType.DMA,
        ],
        compiler_params=pltpu.CompilerParams(needs_layout_passes=False),
    )(pre)
    t32 = out[:, 0:4].reshape(BATCH, 1)
    t64 = out[:, 4:8].reshape(BATCH, 1)
    return t32, t64


def _decode_body(pre_ref, t32_ref, t64_ref, wd_ref, x_ref, bdec_ref,
                 lat_ref, xhat_ref, loss_ref, aux_ref, acc1, acc2):
    i = pl.program_id(0)

    @pl.when(i == 0)
    def _():
        acc1[...] = jnp.zeros_like(acc1)
        acc2[...] = jnp.zeros_like(acc2)

    pre = pre_ref[...]
    u = _skeys(pre)
    relu = jnp.maximum(pre, 0.0)
    m1 = u >= t32_ref[...]
    m2 = (u >= t64_ref[...]) & jnp.logical_not(m1)
    lat = jnp.where(m1, relu, 0.0)
    lat_ref[...] = lat

    wd = wd_ref[...]
    norm2 = jnp.sum(wd * wd, axis=1, keepdims=True)
    inv = 1.0 / jnp.maximum(jnp.sqrt(norm2), 1e-12)
    wdn = wd * inv
    acc1[...] += jnp.dot(lat, wdn, preferred_element_type=jnp.float32)
    aux = jnp.where(m2, relu, 0.0)
    acc2[...] += jnp.dot(aux, wdn, preferred_element_type=jnp.float32)

    @pl.when(i == NBLK - 1)
    def _():
        xh = acc1[...] + bdec_ref[...]
        xhat_ref[...] = xh
        d = xh - x_ref[...]
        loss_ref[0, 0] = jnp.mean(d * d)
        a = acc2[...] + d
        aux_ref[0, 0] = AUX_COEF * jnp.mean(a * a)


def _decode(pre, t32, t64, W_dec, x, bdec2d):
    return pl.pallas_call(
        _decode_body,
        grid=(NBLK,),
        in_specs=[
            pl.BlockSpec((BATCH, BLK), lambda i: (0, i)),
            pl.BlockSpec((BATCH, 1), lambda i: (0, 0)),
            pl.BlockSpec((BATCH, 1), lambda i: (0, 0)),
            pl.BlockSpec((BLK, D_MODEL), lambda i: (i, 0)),
            pl.BlockSpec((BATCH, D_MODEL), lambda i: (0, 0)),
            pl.BlockSpec((1, D_MODEL), lambda i: (0, 0)),
        ],
        out_specs=(
            pl.BlockSpec((BATCH, BLK), lambda i: (0, i)),
            pl.BlockSpec((BATCH, D_MODEL), lambda i: (0, 0)),
            pl.BlockSpec(memory_space=pltpu.SMEM, block_shape=(1, 1),
                         index_map=lambda i: (0, 0)),
            pl.BlockSpec(memory_space=pltpu.SMEM, block_shape=(1, 1),
                         index_map=lambda i: (0, 0)),
        ),
        out_shape=(
            jax.ShapeDtypeStruct((BATCH, D_SAE), jnp.float32),
            jax.ShapeDtypeStruct((BATCH, D_MODEL), jnp.float32),
            jax.ShapeDtypeStruct((1, 1), jnp.float32),
            jax.ShapeDtypeStruct((1, 1), jnp.float32),
        ),
        scratch_shapes=[
            pltpu.VMEM((BATCH, D_MODEL), jnp.float32),
            pltpu.VMEM((BATCH, D_MODEL), jnp.float32),
        ],
    )(pre, t32, t64, W_dec, x, bdec2d)


def kernel(x, W_enc, b_enc, W_dec, b_dec):
    pre = _encode(x, W_enc, b_enc.reshape(1, D_SAE))
    t32, t64 = _thresholds(pre)
    latents, x_hat, loss, aux_loss = _decode(
        pre, t32, t64, W_dec, x, b_dec.reshape(1, D_MODEL))
    return x_hat, latents, loss[0, 0], aux_loss[0, 0]
